# baseline (speedup 1.0000x reference)
import functools

import jax
import jax.numpy as jnp
from jax import lax
from jax.experimental import pallas as pl
from jax.experimental.pallas import tpu as pltpu
from jax.experimental.pallas import tpu_sc as plsc

N_SUBKEYS = 256
D_KEY = 64
TOP_K = 8

TB = 256


_INT_MIN = -2147483648


def _to_sortable(s):
    b = lax.bitcast_convert_type(s, jnp.int32)
    return jnp.where(s < 0, jnp.int32(_INT_MIN) - b, b)


def _from_sortable(k):
    b = jnp.where(k < 0, jnp.int32(_INT_MIN) - k, k)
    return lax.bitcast_convert_type(b, jnp.float32)


def _topk_packed(key, n):
    out = []
    for _ in range(n):
        m = jnp.max(key, axis=1, keepdims=True)
        out.append(m)
        key = jnp.where(key == m, jnp.int32(_INT_MIN), key)
    return jnp.concatenate(out, axis=1)


def _score_kernel(x_ref, wq_ref, ka_ref, kb_ref, wg_ref, bg_ref,
                  idx_ref, wts_ref):
    x = x_ref[...]
    q = lax.dot_general(x, wq_ref[...], (((1,), (1,)), ((), ())),
                        preferred_element_type=jnp.float32)
    qa = q[:, :D_KEY]
    qb = q[:, D_KEY:]
    sa = lax.dot_general(qa, ka_ref[...], (((1,), (1,)), ((), ())),
                         preferred_element_type=jnp.float32)
    sb = lax.dot_general(qb, kb_ref[...], (((1,), (1,)), ((), ())),
                         preferred_element_type=jnp.float32)

    iota256 = lax.broadcasted_iota(jnp.int32, (TB, N_SUBKEYS), 1)
    inv_iota = jnp.int32(N_SUBKEYS - 1) - iota256
    hi24 = jnp.int32(-256)
    ka_key = (_to_sortable(sa) & hi24) | inv_iota
    kb_key = (_to_sortable(sb) & hi24) | inv_iota
    ta = _topk_packed(ka_key, TOP_K)
    tb = _topk_packed(kb_key, TOP_K)
    ia = jnp.int32(N_SUBKEYS - 1) - (ta & jnp.int32(0xFF))
    ib = jnp.int32(N_SUBKEYS - 1) - (tb & jnp.int32(0xFF))
    va = _from_sortable(ta & hi24)
    vb = _from_sortable(tb & hi24)

    comb = jnp.concatenate([va[:, p:p + 1] + vb for p in range(TOP_K)],
                           axis=1)
    flat = jnp.concatenate([ia[:, p:p + 1] * N_SUBKEYS + ib
                            for p in range(TOP_K)], axis=1)
    hi16 = jnp.int32(-65536)
    key2 = ((_to_sortable(comb) & hi16) | (jnp.int32(0xFFFF) - flat))
    t2 = _topk_packed(key2, TOP_K)
    fidx = jnp.int32(0xFFFF) - (t2 & jnp.int32(0xFFFF))
    fvals = _from_sortable(t2 & hi16)

    e = jnp.exp(fvals - fvals[:, 0:1])
    denom = jnp.sum(e, axis=1, keepdims=True)

    glin = jnp.sum(x * wg_ref[...], axis=1, keepdims=True) + bg_ref[0, 0]
    g = jax.nn.sigmoid(glin)
    scale = g / denom

    idx_ref[...] = fidx
    w8 = e * scale
    wts_ref[...] = jnp.concatenate(
        [jnp.broadcast_to(w8[:, k:k + 1], (TB, 16)) for k in range(TOP_K)],
        axis=1)


def _run_score(x2, wq, ka, kb, wg, bg):
    n_tok = x2.shape[0]
    d = x2.shape[1]
    grid = n_tok // TB
    return pl.pallas_call(
        _score_kernel,
        grid=(grid,),
        in_specs=[
            pl.BlockSpec((TB, d), lambda i: (i, 0)),
            pl.BlockSpec((2 * D_KEY, d), lambda i: (0, 0)),
            pl.BlockSpec((N_SUBKEYS, D_KEY), lambda i: (0, 0)),
            pl.BlockSpec((N_SUBKEYS, D_KEY), lambda i: (0, 0)),
            pl.BlockSpec((1, d), lambda i: (0, 0)),
            pl.BlockSpec((1, 1), lambda i: (0, 0)),
        ],
        out_specs=[
            pl.BlockSpec((TB, TOP_K), lambda i: (i, 0)),
            pl.BlockSpec((TB, 16 * TOP_K), lambda i: (i, 0)),
        ],
        out_shape=[
            jax.ShapeDtypeStruct((n_tok, TOP_K), jnp.int32),
            jax.ShapeDtypeStruct((n_tok, 16 * TOP_K), jnp.float32),
        ],
    )(x2, wq, ka, kb, wg, bg)


C = 4
ROWS_C = C * TOP_K


def _sc_body(n_tok, d, values_hbm, idx_hbm, w_hbm, x_hbm, out_hbm,
             idx_v, w_v, rows_v, x_v, out_v,
             gsem0, gsem1, xsem0, xsem1, osem0, osem1):
    info = plsc.get_sparse_core_info()
    nc = info.num_cores
    tpt = n_tok // (nc * info.num_subcores)
    wid = lax.axis_index("s") * nc + lax.axis_index("c")
    base_tok = wid * tpt
    gsem = (gsem0, gsem1)
    xsem = (xsem0, xsem1)
    osem = (osem0, osem1)

    pltpu.sync_copy(idx_hbm.at[pl.ds(base_tok * TOP_K, tpt * TOP_K)], idx_v)
    pltpu.sync_copy(w_hbm.at[pl.ds(base_tok, tpt)], w_v)

    n_chunks = tpt // C
    nd = d // 16

    def issue(c, slot):
        tok0 = base_tok + c * C
        pltpu.async_copy(values_hbm.at[idx_v.at[pl.ds(c * ROWS_C, ROWS_C)]],
                         rows_v.at[slot], gsem[slot])
        pltpu.async_copy(x_hbm.at[pl.ds(tok0, C)], x_v.at[slot], xsem[slot])

    def wait_in(slot):
        pltpu.make_async_copy(values_hbm.at[pl.ds(0, ROWS_C)],
                              rows_v.at[slot], gsem[slot]).wait()
        pltpu.make_async_copy(x_hbm.at[pl.ds(0, C)], x_v.at[slot],
                              xsem[slot]).wait()

    def wait_out(slot):
        pltpu.make_async_copy(out_v.at[slot], out_hbm.at[pl.ds(0, C)],
                              osem[slot]).wait()

    def compute_store(c, slot):
        tok0 = base_tok + c * C
        for t in range(C):
            wvecs = [w_v[c * C + t, pl.ds(16 * k, 16)] for k in range(TOP_K)]

            def dblk(b, _):
                off = pl.ds(b * 16, 16)
                acc = x_v[slot, t, off]
                for k in range(TOP_K):
                    acc = acc + wvecs[k] * rows_v[slot, t * TOP_K + k, off]
                out_v[slot, t, off] = acc
                return 0

            lax.fori_loop(0, nd, dblk, 0, unroll=8)
        pltpu.async_copy(out_v.at[slot], out_hbm.at[pl.ds(tok0, C)],
                         osem[slot])

    issue(0, 0)
    issue(1, 1)
    for slot in (0, 1):
        wait_in(slot)
        compute_store(slot, slot)
        issue(slot + 2, slot)

    def steady(m, _):
        for slot in (0, 1):
            c = 2 * m + slot
            wait_in(slot)
            wait_out(slot)
            compute_store(c, slot)
            issue(c + 2, slot)
        return 0

    lax.fori_loop(1, n_chunks // 2 - 1, steady, 0)

    for slot in (0, 1):
        c = n_chunks - 2 + slot
        wait_in(slot)
        wait_out(slot)
        compute_store(c, slot)
    for slot in (0, 1):
        wait_out(slot)


def _run_sc(values, idx_flat, w_flat, x2):
    n_tok, d = x2.shape
    mesh = plsc.VectorSubcoreMesh(core_axis_name="c", subcore_axis_name="s")
    body = functools.partial(_sc_body, n_tok, d)
    kern = pl.kernel(
        body,
        out_type=jax.ShapeDtypeStruct((n_tok, d), jnp.float32),
        mesh=mesh,
        scratch_types=[
            pltpu.VMEM((n_tok // 32 * TOP_K,), jnp.int32),
            pltpu.VMEM((n_tok // 32, 16 * TOP_K), jnp.float32),
            pltpu.VMEM((2, ROWS_C, d), jnp.float32),
            pltpu.VMEM((2, C, d), jnp.float32),
            pltpu.VMEM((2, C, d), jnp.float32),
            pltpu.SemaphoreType.DMA,
            pltpu.SemaphoreType.DMA,
            pltpu.SemaphoreType.DMA,
            pltpu.SemaphoreType.DMA,
            pltpu.SemaphoreType.DMA,
            pltpu.SemaphoreType.DMA,
        ],
    )
    return kern(values, idx_flat, w_flat, x2)


def kernel(x, keys_a, keys_b, values, W_q, W_g, b_g):
    B, T, D = x.shape
    x2 = x.reshape(B * T, D)
    idx, wts = _run_score(x2, W_q, keys_a, keys_b, W_g,
                          b_g.reshape(1, 1))
    out = _run_sc(values, idx.reshape(-1), wts, x2)
    return out.reshape(B, T, D)
